# baseline (speedup 1.0000x reference)
import functools

import jax
import jax.numpy as jnp
from jax import lax
from jax.experimental import pallas as pl
from jax.experimental.pallas import tpu as pltpu
from jax.experimental.pallas import tpu_sc as plsc

N = 10000
NPAD = 10240
D = 128
E = 320000
K = 64
CH = 160
NW = 32
EPAD = NW * CH * K
RPS = NPAD // 16
G = 40
NB = 4
KD = 128
CHD = EPAD // (NW * KD)
ROWB = 1024


@functools.cache
def _deg_kernel():
    mesh = plsc.VectorSubcoreMesh(core_axis_name="c", subcore_axis_name="s")

    @functools.partial(
        pl.kernel,
        mesh=mesh,
        out_type=jax.ShapeDtypeStruct((2, NPAD), jnp.float32),
        scratch_types=[
            pltpu.VMEM((CHD, KD), jnp.int32),
            pltpu.VMEM((KD,), jnp.float32),
            pltpu.VMEM((RPS,), jnp.float32),
            pltpu.VMEM_SHARED((NPAD,), jnp.float32),
            pltpu.SemaphoreType.DMA,
        ],
    )
    def deg_k(dst_hbm, out_hbm, dstv, ones_v, zb, acc, sem):
        c = lax.axis_index("c")
        s = lax.axis_index("s")
        wid = s * 2 + c

        for t in range(KD // 16):
            ones_v[pl.ds(t * 16, 16)] = jnp.ones((16,), jnp.float32)

        def zfill(i, carry):
            zb[pl.ds(i * 16, 16)] = jnp.zeros((16,), jnp.float32)
            return carry

        lax.fori_loop(0, RPS // 16, zfill, 0)
        pltpu.sync_copy(zb, acc.at[pl.ds(s * RPS, RPS)])
        plsc.subcore_barrier()

        pltpu.sync_copy(dst_hbm.at[pl.ds(wid * CHD, CHD)], dstv)

        def body(g, carry):
            handles = [
                pltpu.async_copy(ones_v, acc.at[dstv.at[8 * g + b]], sem,
                                 add=True)
                for b in range(8)
            ]
            for h in handles:
                h.wait()
            return carry

        lax.fori_loop(0, CHD // 8, body, 0)
        plsc.subcore_barrier()
        pltpu.sync_copy(acc.at[pl.ds(s * RPS, RPS)],
                        out_hbm.at[c, pl.ds(s * RPS, RPS)])

    return deg_k


@functools.cache
def _edge_kernel():
    mesh = plsc.VectorSubcoreMesh(core_axis_name="c", subcore_axis_name="s")

    @functools.partial(
        pl.kernel,
        mesh=mesh,
        out_type=jax.ShapeDtypeStruct((2, NPAD, D), jnp.float32),
        scratch_types=[
            pltpu.VMEM((G, K), jnp.int32),
            pltpu.VMEM((G, K), jnp.int32),
            pltpu.VMEM((NB, K, D), jnp.float32),
            pltpu.VMEM_SHARED((NPAD, D), jnp.float32),
            pltpu.SemaphoreType.DMA,
            pltpu.SemaphoreType.DMA,
            pltpu.SemaphoreType.DMA,
            pltpu.SemaphoreType.DMA,
        ],
    )
    def edge_k(hs_hbm, src_hbm, dst_hbm, out_hbm, srcv, dstv, rows, acc,
               sm0, sm1, sm2, sm3):
        c = lax.axis_index("c")
        s = lax.axis_index("s")
        wid = s * 2 + c
        sems = [sm0, sm1, sm2, sm3]

        def zrow(i, carry):
            for t in range(D // 16):
                rows[0, i, pl.ds(t * 16, 16)] = jnp.zeros((16,), jnp.float32)
            return carry

        lax.fori_loop(0, K, zrow, 0)
        for t in range(RPS // K):
            pltpu.sync_copy(rows.at[0], acc.at[pl.ds(s * RPS + t * K, K)])
        plsc.subcore_barrier()

        def wait_gather(b):
            pltpu.make_async_copy(
                hs_hbm.at[pl.ds(0, K)], rows.at[b], sems[b]).wait()

        for grp in range(CH // G):
            base = wid * CH + grp * G
            pltpu.sync_copy(src_hbm.at[pl.ds(base, G)], srcv)
            pltpu.sync_copy(dst_hbm.at[pl.ds(base, G)], dstv)
            for b in range(NB):
                pltpu.async_copy(hs_hbm.at[srcv.at[b]], rows.at[b], sems[b])

            def quad(t, carry):
                for b in range(NB):
                    j = NB * t + b
                    wait_gather(b)
                    pltpu.sync_copy(rows.at[b], acc.at[dstv.at[j]], add=True)
                    pltpu.async_copy(
                        hs_hbm.at[srcv.at[j + NB]], rows.at[b], sems[b])
                return carry

            lax.fori_loop(0, G // NB - 1, quad, 0)
            for b in range(NB):
                wait_gather(b)
                pltpu.sync_copy(rows.at[b], acc.at[dstv.at[G - NB + b]],
                                add=True)
        plsc.subcore_barrier()
        handles = [
            pltpu.async_copy(acc.at[pl.ds(s * RPS + t * K, K)],
                             out_hbm.at[c, pl.ds(s * RPS + t * K, K)],
                             sems[t % NB])
            for t in range(RPS // K)
        ]
        for h in handles:
            h.wait()

    return edge_k


def _deg_finalize(degp_g):
    def body(d_ref, o_ref):
        o_ref[...] = lax.rsqrt(d_ref[0] + d_ref[1] + 1.0)

    return pl.pallas_call(
        body,
        out_shape=jax.ShapeDtypeStruct((NPAD // 128, 128), jnp.float32),
    )(degp_g)


def _scale_rows(x, dinv_b):
    def body(x_ref, d_ref, o_ref):
        o_ref[...] = x_ref[...] * d_ref[...]

    return pl.pallas_call(
        body,
        grid=(NPAD // ROWB,),
        in_specs=[
            pl.BlockSpec((ROWB, D), lambda i: (i, 0)),
            pl.BlockSpec((ROWB, D), lambda i: (i, 0)),
        ],
        out_specs=pl.BlockSpec((ROWB, D), lambda i: (i, 0)),
        out_shape=jax.ShapeDtypeStruct((NPAD, D), jnp.float32),
    )(x, dinv_b)


def _mid_layer(accp, xs, dinv_b, W1, b1):
    def body(a_ref, h_ref, d_ref, w_ref, b_ref, o_ref):
        dinv = d_ref[...]
        z = (a_ref[0] + a_ref[1] + h_ref[...]) * dinv
        h1 = jnp.maximum(
            jnp.dot(z, w_ref[...], preferred_element_type=jnp.float32)
            + b_ref[...], 0.0)
        o_ref[...] = h1 * dinv

    return pl.pallas_call(
        body,
        grid=(NPAD // ROWB,),
        in_specs=[
            pl.BlockSpec((2, ROWB, D), lambda i: (0, i, 0)),
            pl.BlockSpec((ROWB, D), lambda i: (i, 0)),
            pl.BlockSpec((ROWB, D), lambda i: (i, 0)),
            pl.BlockSpec((D, D), lambda i: (0, 0)),
            pl.BlockSpec((1, D), lambda i: (0, 0)),
        ],
        out_specs=pl.BlockSpec((ROWB, D), lambda i: (i, 0)),
        out_shape=jax.ShapeDtypeStruct((NPAD, D), jnp.float32),
    )(accp, xs, dinv_b, W1, b1)


def _final_layer(accp, hs, dinv_b, W2, b2):
    def body(a_ref, h_ref, d_ref, w_ref, b_ref, o_ref):
        z = (a_ref[0] + a_ref[1] + h_ref[...]) * d_ref[...]
        o_ref[...] = jnp.dot(
            z, w_ref[...], preferred_element_type=jnp.float32) + b_ref[...]

    return pl.pallas_call(
        body,
        grid=(NPAD // ROWB,),
        in_specs=[
            pl.BlockSpec((2, ROWB, D), lambda i: (0, i, 0)),
            pl.BlockSpec((ROWB, D), lambda i: (i, 0)),
            pl.BlockSpec((ROWB, D), lambda i: (i, 0)),
            pl.BlockSpec((D, D), lambda i: (0, 0)),
            pl.BlockSpec((1, D), lambda i: (0, 0)),
        ],
        out_specs=pl.BlockSpec((ROWB, D), lambda i: (i, 0)),
        out_shape=jax.ShapeDtypeStruct((NPAD, D), jnp.float32),
    )(accp, hs, dinv_b, W2, b2)


def kernel(x, edge_index, W1, b1, W2, b2):
    src = edge_index[0].astype(jnp.int32)
    dst = edge_index[1].astype(jnp.int32)

    n_extra = EPAD - E
    pad_rows = N + (jnp.arange(n_extra, dtype=jnp.int32) % (NPAD - N))
    src_p = jnp.concatenate([src, pad_rows])
    dst_p = jnp.concatenate([dst, pad_rows])
    src2d = src_p.reshape(NW * CH, K)
    dst2d = dst_p.reshape(NW * CH, K)

    degp = _deg_kernel()(dst_p.reshape(NW * CHD, KD))
    dinv_g = _deg_finalize(degp.reshape(2, NPAD // 128, 128))
    dinv_b = jnp.broadcast_to(dinv_g.reshape(NPAD, 1), (NPAD, D))

    xs = _scale_rows(x, dinv_b)
    accp1 = _edge_kernel()(xs, src2d, dst2d)
    h1s = _mid_layer(accp1, xs, dinv_b, W1, b1.reshape(1, D))
    accp2 = _edge_kernel()(h1s, src2d, dst2d)
    out = _final_layer(accp2, h1s, dinv_b, W2, b2.reshape(1, D))
    return out[:N]


# speedup vs baseline: 1.0253x; 1.0253x over previous
import functools

import jax
import jax.numpy as jnp
from jax import lax
from jax.experimental import pallas as pl
from jax.experimental.pallas import tpu as pltpu
from jax.experimental.pallas import tpu_sc as plsc

N = 10000
NPAD = 10240
D = 128
E = 320000
K = 64
CH = 160
NW = 32
EPAD = NW * CH * K
RPS = NPAD // 16
G = 40
NB = 4
KD = 128
CHD = EPAD // (NW * KD)
ROWB = 2048


@functools.cache
def _deg_kernel():
    mesh = plsc.VectorSubcoreMesh(core_axis_name="c", subcore_axis_name="s")

    @functools.partial(
        pl.kernel,
        mesh=mesh,
        out_type=jax.ShapeDtypeStruct((2, NPAD), jnp.float32),
        scratch_types=[
            pltpu.VMEM((CHD, KD), jnp.int32),
            pltpu.VMEM((KD,), jnp.float32),
            pltpu.VMEM((RPS,), jnp.float32),
            pltpu.VMEM_SHARED((NPAD,), jnp.float32),
            pltpu.SemaphoreType.DMA,
        ],
    )
    def deg_k(dst_hbm, out_hbm, dstv, ones_v, zb, acc, sem):
        c = lax.axis_index("c")
        s = lax.axis_index("s")
        wid = s * 2 + c

        for t in range(KD // 16):
            ones_v[pl.ds(t * 16, 16)] = jnp.ones((16,), jnp.float32)

        def zfill(i, carry):
            zb[pl.ds(i * 16, 16)] = jnp.zeros((16,), jnp.float32)
            return carry

        lax.fori_loop(0, RPS // 16, zfill, 0)
        pltpu.sync_copy(zb, acc.at[pl.ds(s * RPS, RPS)])
        plsc.subcore_barrier()

        pltpu.sync_copy(dst_hbm.at[pl.ds(wid * CHD, CHD)], dstv)

        def body(g, carry):
            handles = [
                pltpu.async_copy(ones_v, acc.at[dstv.at[8 * g + b]], sem,
                                 add=True)
                for b in range(8)
            ]
            for h in handles:
                h.wait()
            return carry

        lax.fori_loop(0, CHD // 8, body, 0)
        plsc.subcore_barrier()
        pltpu.sync_copy(acc.at[pl.ds(s * RPS, RPS)],
                        out_hbm.at[c, pl.ds(s * RPS, RPS)])

    return deg_k


@functools.cache
def _edge_kernel():
    mesh = plsc.VectorSubcoreMesh(core_axis_name="c", subcore_axis_name="s")

    @functools.partial(
        pl.kernel,
        mesh=mesh,
        out_type=jax.ShapeDtypeStruct((2, NPAD, D), jnp.float32),
        scratch_types=[
            pltpu.VMEM((G, K), jnp.int32),
            pltpu.VMEM((G, K), jnp.int32),
            pltpu.VMEM((NB, K, D), jnp.float32),
            pltpu.VMEM_SHARED((NPAD, D), jnp.float32),
            pltpu.SemaphoreType.DMA,
            pltpu.SemaphoreType.DMA,
            pltpu.SemaphoreType.DMA,
            pltpu.SemaphoreType.DMA,
        ],
    )
    def edge_k(hs_hbm, src_hbm, dst_hbm, out_hbm, srcv, dstv, rows, acc,
               sm0, sm1, sm2, sm3):
        c = lax.axis_index("c")
        s = lax.axis_index("s")
        wid = s * 2 + c
        sems = [sm0, sm1, sm2, sm3]

        def zrow(i, carry):
            for t in range(D // 16):
                rows[0, i, pl.ds(t * 16, 16)] = jnp.zeros((16,), jnp.float32)
            return carry

        lax.fori_loop(0, K, zrow, 0)
        for t in range(RPS // K):
            pltpu.sync_copy(rows.at[0], acc.at[pl.ds(s * RPS + t * K, K)])
        plsc.subcore_barrier()

        def wait_gather(b):
            pltpu.make_async_copy(
                hs_hbm.at[pl.ds(0, K)], rows.at[b], sems[b]).wait()

        for grp in range(CH // G):
            base = wid * CH + grp * G
            pltpu.sync_copy(src_hbm.at[pl.ds(base, G)], srcv)
            pltpu.sync_copy(dst_hbm.at[pl.ds(base, G)], dstv)
            for b in range(NB):
                pltpu.async_copy(hs_hbm.at[srcv.at[b]], rows.at[b], sems[b])

            def quad(t, carry):
                for b in range(NB):
                    j = NB * t + b
                    wait_gather(b)
                    pltpu.sync_copy(rows.at[b], acc.at[dstv.at[j]], add=True)
                    pltpu.async_copy(
                        hs_hbm.at[srcv.at[j + NB]], rows.at[b], sems[b])
                return carry

            lax.fori_loop(0, G // NB - 1, quad, 0)
            for b in range(NB):
                wait_gather(b)
                pltpu.sync_copy(rows.at[b], acc.at[dstv.at[G - NB + b]],
                                add=True)
        plsc.subcore_barrier()
        handles = [
            pltpu.async_copy(acc.at[pl.ds(s * RPS + t * K, K)],
                             out_hbm.at[c, pl.ds(s * RPS + t * K, K)],
                             sems[t % NB])
            for t in range(RPS // K)
        ]
        for h in handles:
            h.wait()

    return edge_k


def _deg_finalize(degp_g):
    def body(d_ref, o_ref):
        o_ref[...] = lax.rsqrt(d_ref[0] + d_ref[1] + 1.0)

    return pl.pallas_call(
        body,
        out_shape=jax.ShapeDtypeStruct((NPAD // 128, 128), jnp.float32),
    )(degp_g)


def _scale_rows(x, dinv_b):
    def body(x_ref, d_ref, o_ref):
        o_ref[...] = x_ref[...] * d_ref[...]

    return pl.pallas_call(
        body,
        grid=(NPAD // ROWB,),
        in_specs=[
            pl.BlockSpec((ROWB, D), lambda i: (i, 0)),
            pl.BlockSpec((ROWB, D), lambda i: (i, 0)),
        ],
        out_specs=pl.BlockSpec((ROWB, D), lambda i: (i, 0)),
        out_shape=jax.ShapeDtypeStruct((NPAD, D), jnp.float32),
    )(x, dinv_b)


def _mid_layer(accp, xs, dinv_b, W1, b1):
    def body(a_ref, h_ref, d_ref, w_ref, b_ref, o_ref):
        dinv = d_ref[...]
        z = (a_ref[0] + a_ref[1] + h_ref[...]) * dinv
        h1 = jnp.maximum(
            jnp.dot(z, w_ref[...], preferred_element_type=jnp.float32)
            + b_ref[...], 0.0)
        o_ref[...] = h1 * dinv

    return pl.pallas_call(
        body,
        grid=(NPAD // ROWB,),
        in_specs=[
            pl.BlockSpec((2, ROWB, D), lambda i: (0, i, 0)),
            pl.BlockSpec((ROWB, D), lambda i: (i, 0)),
            pl.BlockSpec((ROWB, D), lambda i: (i, 0)),
            pl.BlockSpec((D, D), lambda i: (0, 0)),
            pl.BlockSpec((1, D), lambda i: (0, 0)),
        ],
        out_specs=pl.BlockSpec((ROWB, D), lambda i: (i, 0)),
        out_shape=jax.ShapeDtypeStruct((NPAD, D), jnp.float32),
    )(accp, xs, dinv_b, W1, b1)


def _final_layer(accp, hs, dinv_b, W2, b2):
    def body(a_ref, h_ref, d_ref, w_ref, b_ref, o_ref):
        z = (a_ref[0] + a_ref[1] + h_ref[...]) * d_ref[...]
        o_ref[...] = jnp.dot(
            z, w_ref[...], preferred_element_type=jnp.float32) + b_ref[...]

    return pl.pallas_call(
        body,
        grid=(NPAD // ROWB,),
        in_specs=[
            pl.BlockSpec((2, ROWB, D), lambda i: (0, i, 0)),
            pl.BlockSpec((ROWB, D), lambda i: (i, 0)),
            pl.BlockSpec((ROWB, D), lambda i: (i, 0)),
            pl.BlockSpec((D, D), lambda i: (0, 0)),
            pl.BlockSpec((1, D), lambda i: (0, 0)),
        ],
        out_specs=pl.BlockSpec((ROWB, D), lambda i: (i, 0)),
        out_shape=jax.ShapeDtypeStruct((NPAD, D), jnp.float32),
    )(accp, hs, dinv_b, W2, b2)


def kernel(x, edge_index, W1, b1, W2, b2):
    src = edge_index[0].astype(jnp.int32)
    dst = edge_index[1].astype(jnp.int32)

    n_extra = EPAD - E
    pad_rows = N + (jnp.arange(n_extra, dtype=jnp.int32) % (NPAD - N))
    src_p = jnp.concatenate([src, pad_rows])
    dst_p = jnp.concatenate([dst, pad_rows])
    src2d = src_p.reshape(NW * CH, K)
    dst2d = dst_p.reshape(NW * CH, K)

    degp = _deg_kernel()(dst_p.reshape(NW * CHD, KD))
    dinv_g = _deg_finalize(degp.reshape(2, NPAD // 128, 128))
    dinv_b = jnp.broadcast_to(dinv_g.reshape(NPAD, 1), (NPAD, D))

    xs = _scale_rows(x, dinv_b)
    accp1 = _edge_kernel()(xs, src2d, dst2d)
    h1s = _mid_layer(accp1, xs, dinv_b, W1, b1.reshape(1, D))
    accp2 = _edge_kernel()(h1s, src2d, dst2d)
    out = _final_layer(accp2, h1s, dinv_b, W2, b2.reshape(1, D))
    return out[:N]


# speedup vs baseline: 1.0273x; 1.0019x over previous
import functools

import jax
import jax.numpy as jnp
from jax import lax
from jax.experimental import pallas as pl
from jax.experimental.pallas import tpu as pltpu
from jax.experimental.pallas import tpu_sc as plsc

N = 10000
NPAD = 10240
D = 128
E = 320000
K = 64
CH = 160
NW = 32
EPAD = NW * CH * K
RPS = NPAD // 16
G = 40
NB = 4
KD = 128
CHD = EPAD // (NW * KD)
ROWB = 2048


@functools.cache
def _deg_kernel():
    mesh = plsc.VectorSubcoreMesh(core_axis_name="c", subcore_axis_name="s")

    @functools.partial(
        pl.kernel,
        mesh=mesh,
        out_type=jax.ShapeDtypeStruct((2, NPAD), jnp.float32),
        scratch_types=[
            pltpu.VMEM((CHD, KD), jnp.int32),
            pltpu.VMEM((KD,), jnp.float32),
            pltpu.VMEM((RPS,), jnp.float32),
            pltpu.VMEM_SHARED((NPAD,), jnp.float32),
            pltpu.SemaphoreType.DMA,
        ],
    )
    def deg_k(dst_hbm, out_hbm, dstv, ones_v, zb, acc, sem):
        c = lax.axis_index("c")
        s = lax.axis_index("s")
        wid = s * 2 + c

        for t in range(KD // 16):
            ones_v[pl.ds(t * 16, 16)] = jnp.ones((16,), jnp.float32)

        def zfill(i, carry):
            zb[pl.ds(i * 16, 16)] = jnp.zeros((16,), jnp.float32)
            return carry

        lax.fori_loop(0, RPS // 16, zfill, 0)
        pltpu.sync_copy(zb, acc.at[pl.ds(s * RPS, RPS)])
        plsc.subcore_barrier()

        pltpu.sync_copy(dst_hbm.at[pl.ds(wid * CHD, CHD)], dstv)

        def body(g, carry):
            handles = [
                pltpu.async_copy(ones_v, acc.at[dstv.at[16 * g + b]], sem,
                                 add=True)
                for b in range(16)
            ]
            for h in handles:
                h.wait()
            return carry

        lax.fori_loop(0, CHD // 16, body, 0)
        plsc.subcore_barrier()
        pltpu.sync_copy(acc.at[pl.ds(s * RPS, RPS)],
                        out_hbm.at[c, pl.ds(s * RPS, RPS)])

    return deg_k


@functools.cache
def _edge_kernel():
    mesh = plsc.VectorSubcoreMesh(core_axis_name="c", subcore_axis_name="s")

    @functools.partial(
        pl.kernel,
        mesh=mesh,
        out_type=jax.ShapeDtypeStruct((2, NPAD, D), jnp.float32),
        scratch_types=[
            pltpu.VMEM((G, K), jnp.int32),
            pltpu.VMEM((G, K), jnp.int32),
            pltpu.VMEM((NB, K, D), jnp.float32),
            pltpu.VMEM_SHARED((NPAD, D), jnp.float32),
            pltpu.SemaphoreType.DMA,
            pltpu.SemaphoreType.DMA,
            pltpu.SemaphoreType.DMA,
            pltpu.SemaphoreType.DMA,
        ],
    )
    def edge_k(hs_hbm, src_hbm, dst_hbm, out_hbm, srcv, dstv, rows, acc,
               sm0, sm1, sm2, sm3):
        c = lax.axis_index("c")
        s = lax.axis_index("s")
        wid = s * 2 + c
        sems = [sm0, sm1, sm2, sm3]

        def zrow(i, carry):
            for t in range(D // 16):
                rows[0, i, pl.ds(t * 16, 16)] = jnp.zeros((16,), jnp.float32)
            return carry

        lax.fori_loop(0, K, zrow, 0)
        zh = [
            pltpu.async_copy(rows.at[0], acc.at[pl.ds(s * RPS + t * K, K)],
                             sems[t % NB])
            for t in range(RPS // K)
        ]
        for h in zh:
            h.wait()
        plsc.subcore_barrier()

        def wait_gather(b):
            pltpu.make_async_copy(
                hs_hbm.at[pl.ds(0, K)], rows.at[b], sems[b]).wait()

        for grp in range(CH // G):
            base = wid * CH + grp * G
            pltpu.sync_copy(src_hbm.at[pl.ds(base, G)], srcv)
            pltpu.sync_copy(dst_hbm.at[pl.ds(base, G)], dstv)
            for b in range(NB):
                pltpu.async_copy(hs_hbm.at[srcv.at[b]], rows.at[b], sems[b])

            def quad(t, carry):
                for b in range(NB):
                    j = NB * t + b
                    wait_gather(b)
                    pltpu.sync_copy(rows.at[b], acc.at[dstv.at[j]], add=True)
                    pltpu.async_copy(
                        hs_hbm.at[srcv.at[j + NB]], rows.at[b], sems[b])
                return carry

            lax.fori_loop(0, G // NB - 1, quad, 0)
            for b in range(NB):
                wait_gather(b)
                pltpu.sync_copy(rows.at[b], acc.at[dstv.at[G - NB + b]],
                                add=True)
        plsc.subcore_barrier()
        handles = [
            pltpu.async_copy(acc.at[pl.ds(s * RPS + t * K, K)],
                             out_hbm.at[c, pl.ds(s * RPS + t * K, K)],
                             sems[t % NB])
            for t in range(RPS // K)
        ]
        for h in handles:
            h.wait()

    return edge_k


def _deg_finalize(degp_g):
    def body(d_ref, o_ref):
        o_ref[...] = lax.rsqrt(d_ref[0] + d_ref[1] + 1.0)

    return pl.pallas_call(
        body,
        out_shape=jax.ShapeDtypeStruct((NPAD // 128, 128), jnp.float32),
    )(degp_g)


def _scale_rows(x, dinv_b):
    def body(x_ref, d_ref, o_ref):
        o_ref[...] = x_ref[...] * d_ref[...]

    return pl.pallas_call(
        body,
        grid=(NPAD // ROWB,),
        in_specs=[
            pl.BlockSpec((ROWB, D), lambda i: (i, 0)),
            pl.BlockSpec((ROWB, D), lambda i: (i, 0)),
        ],
        out_specs=pl.BlockSpec((ROWB, D), lambda i: (i, 0)),
        out_shape=jax.ShapeDtypeStruct((NPAD, D), jnp.float32),
    )(x, dinv_b)


def _mid_layer(accp, xs, dinv_b, W1, b1):
    def body(a_ref, h_ref, d_ref, w_ref, b_ref, o_ref):
        dinv = d_ref[...]
        z = (a_ref[0] + a_ref[1] + h_ref[...]) * dinv
        h1 = jnp.maximum(
            jnp.dot(z, w_ref[...], preferred_element_type=jnp.float32)
            + b_ref[...], 0.0)
        o_ref[...] = h1 * dinv

    return pl.pallas_call(
        body,
        grid=(NPAD // ROWB,),
        in_specs=[
            pl.BlockSpec((2, ROWB, D), lambda i: (0, i, 0)),
            pl.BlockSpec((ROWB, D), lambda i: (i, 0)),
            pl.BlockSpec((ROWB, D), lambda i: (i, 0)),
            pl.BlockSpec((D, D), lambda i: (0, 0)),
            pl.BlockSpec((1, D), lambda i: (0, 0)),
        ],
        out_specs=pl.BlockSpec((ROWB, D), lambda i: (i, 0)),
        out_shape=jax.ShapeDtypeStruct((NPAD, D), jnp.float32),
    )(accp, xs, dinv_b, W1, b1)


def _final_layer(accp, hs, dinv_b, W2, b2):
    def body(a_ref, h_ref, d_ref, w_ref, b_ref, o_ref):
        z = (a_ref[0] + a_ref[1] + h_ref[...]) * d_ref[...]
        o_ref[...] = jnp.dot(
            z, w_ref[...], preferred_element_type=jnp.float32) + b_ref[...]

    return pl.pallas_call(
        body,
        grid=(NPAD // ROWB,),
        in_specs=[
            pl.BlockSpec((2, ROWB, D), lambda i: (0, i, 0)),
            pl.BlockSpec((ROWB, D), lambda i: (i, 0)),
            pl.BlockSpec((ROWB, D), lambda i: (i, 0)),
            pl.BlockSpec((D, D), lambda i: (0, 0)),
            pl.BlockSpec((1, D), lambda i: (0, 0)),
        ],
        out_specs=pl.BlockSpec((ROWB, D), lambda i: (i, 0)),
        out_shape=jax.ShapeDtypeStruct((NPAD, D), jnp.float32),
    )(accp, hs, dinv_b, W2, b2)


def kernel(x, edge_index, W1, b1, W2, b2):
    src = edge_index[0].astype(jnp.int32)
    dst = edge_index[1].astype(jnp.int32)

    n_extra = EPAD - E
    pad_rows = N + (jnp.arange(n_extra, dtype=jnp.int32) % (NPAD - N))
    src_p = jnp.concatenate([src, pad_rows])
    dst_p = jnp.concatenate([dst, pad_rows])
    src2d = src_p.reshape(NW * CH, K)
    dst2d = dst_p.reshape(NW * CH, K)

    degp = _deg_kernel()(dst_p.reshape(NW * CHD, KD))
    dinv_g = _deg_finalize(degp.reshape(2, NPAD // 128, 128))
    dinv_b = jnp.broadcast_to(dinv_g.reshape(NPAD, 1), (NPAD, D))

    xs = _scale_rows(x, dinv_b)
    accp1 = _edge_kernel()(xs, src2d, dst2d)
    h1s = _mid_layer(accp1, xs, dinv_b, W1, b1.reshape(1, D))
    accp2 = _edge_kernel()(h1s, src2d, dst2d)
    out = _final_layer(accp2, h1s, dinv_b, W2, b2.reshape(1, D))
    return out[:N]
